# Initial kernel scaffold; baseline (speedup 1.0000x reference)
import jax
import jax.numpy as jnp
from jax.experimental import pallas as pl


def kernel(x, edge_index, Wl0, bl0, Wr0, Wl1, bl1, Wr1, Wm1, bm1, Wm2, bm2):
    raise NotImplementedError("write your pallas kernel here")


# speedup vs baseline: 9.8549x; 9.8549x over previous
import functools

import jax
import jax.numpy as jnp
from jax import lax
from jax.experimental import pallas as pl
from jax.experimental.pallas import tpu as pltpu
from jax.experimental.pallas import tpu_sc as plsc

N = 10000
D = 128
NC = 2
NS = 16
L = 16
NW = NC * NS
CHUNK = 128
IBLK = 8
NPAD = 10240
RPT = NPAD // NS
NKW = RPT // CHUNK


@functools.cache
def _make_agg(nch):
    mesh = plsc.VectorSubcoreMesh(
        core_axis_name="c", subcore_axis_name="s",
        num_cores=NC, num_subcores=NS)

    @functools.partial(
        pl.kernel,
        out_type=[
            jax.ShapeDtypeStruct((NC, NPAD, D), jnp.float32),
            jax.ShapeDtypeStruct((NC, NPAD), jnp.float32),
        ],
        mesh=mesh,
        scratch_types=[
            pltpu.VMEM((IBLK, CHUNK), jnp.int32),
            pltpu.VMEM((IBLK, CHUNK), jnp.int32),
            pltpu.VMEM((CHUNK, D), jnp.float32),
            pltpu.VMEM((CHUNK, D), jnp.float32),
            pltpu.VMEM((CHUNK,), jnp.float32),
            pltpu.VMEM((RPT,), jnp.float32),
            pltpu.VMEM_SHARED((NPAD, D), jnp.float32),
            pltpu.VMEM_SHARED((NPAD,), jnp.float32),
            pltpu.SemaphoreType.DMA,
            pltpu.SemaphoreType.DMA,
            pltpu.SemaphoreType.DMA,
            pltpu.SemaphoreType.DMA,
        ],
    )
    def agg(h_hbm, src_hbm, dst_hbm, zrows_hbm, zc_hbm, ones_hbm,
            sum_hbm, cnt_hbm,
            src_v, dst_v, rows_v, rows2_v, ones_v, cz_v, acc_s, cntacc_s,
            sem0, sem1, ss0, ss1):
        cid = lax.axis_index("c")
        sid = lax.axis_index("s")
        wid = sid * NC + cid
        pltpu.sync_copy(ones_hbm, ones_v)
        pltpu.sync_copy(zc_hbm, cz_v)
        pltpu.sync_copy(zrows_hbm, rows_v)
        for k in range(NKW):
            sl = pl.ds(sid * RPT + k * CHUNK, CHUNK)
            pltpu.sync_copy(rows_v, acc_s.at[sl])
        pltpu.sync_copy(cz_v, cntacc_s.at[pl.ds(sid * RPT, RPT)])
        plsc.subcore_barrier()

        bufs = (rows_v, rows2_v)
        gsems = (sem0, sem1)
        ssems = (ss0, ss1)

        def gather(j, p):
            pltpu.async_copy(h_hbm.at[src_v.at[j]], bufs[p], gsems[p])

        def gather_wait(j, p):
            pltpu.make_async_copy(h_hbm.at[src_v.at[j]], bufs[p],
                                  gsems[p]).wait()

        def scat(j, p):
            pltpu.async_copy(bufs[p], acc_s.at[dst_v.at[j]], ssems[p],
                             add=True)

        def scat_wait(j, p):
            pltpu.make_async_copy(bufs[p], acc_s.at[dst_v.at[j]],
                                  ssems[p]).wait()

        def block(b, carry):
            pltpu.sync_copy(src_hbm.at[wid, pl.ds(b * IBLK, IBLK)], src_v)
            pltpu.sync_copy(dst_hbm.at[wid, pl.ds(b * IBLK, IBLK)], dst_v)
            gather(0, 0)
            for j in range(IBLK):
                p = j % 2
                gather_wait(j, p)
                scat(j, p)
                pltpu.sync_copy(ones_v, cntacc_s.at[dst_v.at[j]], add=True)
                if j + 1 < IBLK:
                    q = (j + 1) % 2
                    if j >= 1:
                        scat_wait(j - 1, q)
                    gather(j + 1, q)
            if IBLK >= 2:
                scat_wait(IBLK - 2, (IBLK - 2) % 2)
            scat_wait(IBLK - 1, (IBLK - 1) % 2)
            return carry

        lax.fori_loop(0, nch // IBLK, block, 0)
        plsc.subcore_barrier()
        for k in range(NKW):
            sl = pl.ds(sid * RPT + k * CHUNK, CHUNK)
            pltpu.sync_copy(acc_s.at[sl], rows_v)
            pltpu.sync_copy(rows_v, sum_hbm.at[cid, sl])
        slc = pl.ds(sid * RPT, RPT)
        pltpu.sync_copy(cntacc_s.at[slc], cz_v)
        pltpu.sync_copy(cz_v, cnt_hbm.at[cid, slc])

    return agg


def _dense_sage(s0, s1, cnt, h, Wl, bl, Wr):
    BR = 2000
    grid = (N // BR,)

    def body(s0_ref, s1_ref, cnt_ref, h_ref, Wl_ref, bl_ref, Wr_ref,
             o_ref):
        denom = jnp.maximum(cnt_ref[...], 1.0)
        mean = (s0_ref[...] + s1_ref[...]) / denom
        out = (jnp.dot(mean, Wl_ref[...], preferred_element_type=jnp.float32)
               + jnp.dot(h_ref[...], Wr_ref[...],
                         preferred_element_type=jnp.float32)
               + bl_ref[...])
        o_ref[...] = jnp.maximum(out, 0.0)

    rowspec = pl.BlockSpec((BR, D), lambda i: (i, 0))
    cntspec = pl.BlockSpec((BR, 1), lambda i: (i, 0))
    wspec = pl.BlockSpec((D, D), lambda i: (0, 0))
    bspec = pl.BlockSpec((1, D), lambda i: (0, 0))
    return pl.pallas_call(
        body, grid=grid,
        in_specs=[rowspec, rowspec, cntspec, rowspec,
                  wspec, bspec, wspec],
        out_specs=rowspec,
        out_shape=jax.ShapeDtypeStruct((N, D), jnp.float32),
    )(s0, s1, cnt, h, Wl, bl.reshape(1, D), Wr)


def _dense_final(s0, s1, cnt, h, Wl, bl, Wr, Wm1p, bm1p, Wm2p, bm2):
    BR = 2000
    grid = (N // BR,)

    def body(s0_ref, s1_ref, cnt_ref, h_ref, Wl_ref, bl_ref, Wr_ref,
             Wm1_ref, bm1_ref, Wm2_ref, bm2_ref, emb_ref, pred_ref):
        denom = jnp.maximum(cnt_ref[...], 1.0)
        mean = (s0_ref[...] + s1_ref[...]) / denom
        emb = (jnp.dot(mean, Wl_ref[...], preferred_element_type=jnp.float32)
               + jnp.dot(h_ref[...], Wr_ref[...],
                         preferred_element_type=jnp.float32)
               + bl_ref[...])
        emb_ref[...] = emb
        h1 = jnp.maximum(emb, 0.0)
        z = jnp.maximum(
            jnp.dot(h1, Wm1_ref[...], preferred_element_type=jnp.float32)
            + bm1_ref[...], 0.0)
        pred_ref[...] = (
            jnp.dot(z, Wm2_ref[...], preferred_element_type=jnp.float32)
            + bm2_ref[...])

    rowspec = pl.BlockSpec((BR, D), lambda i: (i, 0))
    cntspec = pl.BlockSpec((BR, 1), lambda i: (i, 0))
    wspec = pl.BlockSpec((D, D), lambda i: (0, 0))
    bspec = pl.BlockSpec((1, D), lambda i: (0, 0))
    sspec = pl.BlockSpec((1, 1), lambda i: (0, 0))
    return pl.pallas_call(
        body, grid=grid,
        in_specs=[rowspec, rowspec, cntspec, rowspec,
                  wspec, bspec, wspec, wspec, bspec, wspec, sspec],
        out_specs=[rowspec, rowspec],
        out_shape=[jax.ShapeDtypeStruct((N, D), jnp.float32),
                   jax.ShapeDtypeStruct((N, D), jnp.float32)],
    )(s0, s1, cnt, h, Wl, bl.reshape(1, D), Wr,
      Wm1p, bm1p.reshape(1, D), Wm2p, bm2.reshape(1, 1))


def kernel(x, edge_index, Wl0, bl0, Wr0, Wl1, bl1, Wr1, Wm1, bm1, Wm2, bm2):
    E = edge_index.shape[1]
    src = edge_index[0].astype(jnp.int32)
    dst = edge_index[1].astype(jnp.int32)
    nch = -(-(-(-E // (NW * CHUNK))) // IBLK) * IBLK
    epw = nch * CHUNK
    pad = NW * epw - E
    pad_src = (jnp.arange(pad, dtype=jnp.int32) * 61) % N
    pad_dst = N + (jnp.arange(pad, dtype=jnp.int32) % L)
    srcp = jnp.concatenate([src, pad_src]).reshape(NW, nch, CHUNK)
    dstp = jnp.concatenate([dst, pad_dst]).reshape(NW, nch, CHUNK)

    zrows = jnp.zeros((CHUNK, D), jnp.float32)
    zc = jnp.zeros((RPT,), jnp.float32)
    ones = jnp.ones((CHUNK,), jnp.float32)
    Wm1p = jnp.pad(Wm1, ((0, 0), (0, D - Wm1.shape[1])))
    bm1p = jnp.pad(bm1, (0, D - bm1.shape[0]))
    Wm2p = jnp.pad(Wm2, ((0, D - Wm2.shape[0]), (0, D - Wm2.shape[1])))

    agg = _make_agg(nch)
    sums, cnts = agg(x, srcp, dstp, zrows, zc, ones)
    s0, s1 = sums[0, :N], sums[1, :N]
    cnt = (cnts[0, :N] + cnts[1, :N]).reshape(N, 1)
    h0 = _dense_sage(s0, s1, cnt, x, Wl0, bl0, Wr0)
    sums1, _ = agg(h0, srcp, dstp, zrows, zc, ones)
    t0, t1 = sums1[0, :N], sums1[1, :N]
    emb, pred_wide = _dense_final(t0, t1, cnt, h0, Wl1, bl1, Wr1,
                                  Wm1p, bm1p, Wm2p, bm2)
    return emb, pred_wide[:, :1]


# speedup vs baseline: 12.3397x; 1.2521x over previous
import functools

import jax
import jax.numpy as jnp
from jax import lax
from jax.experimental import pallas as pl
from jax.experimental.pallas import tpu as pltpu
from jax.experimental.pallas import tpu_sc as plsc

N = 10000
D = 128
NC = 2
NS = 16
L = 16
NW = NC * NS
CHUNK = 128
IBLK = 16
NPAD = 10240
RPT = NPAD // NS
NKW = RPT // CHUNK


@functools.cache
def _make_agg(nch):
    mesh = plsc.VectorSubcoreMesh(
        core_axis_name="c", subcore_axis_name="s",
        num_cores=NC, num_subcores=NS)

    @functools.partial(
        pl.kernel,
        out_type=[
            jax.ShapeDtypeStruct((NC, NPAD, D), jnp.float32),
            jax.ShapeDtypeStruct((NC, NPAD), jnp.float32),
        ],
        mesh=mesh,
        scratch_types=[
            pltpu.VMEM((IBLK, CHUNK), jnp.int32),
            pltpu.VMEM((IBLK, CHUNK), jnp.int32),
            pltpu.VMEM((CHUNK, D), jnp.float32),
            pltpu.VMEM((CHUNK, D), jnp.float32),
            pltpu.VMEM((CHUNK,), jnp.float32),
            pltpu.VMEM((RPT,), jnp.float32),
            pltpu.VMEM_SHARED((NPAD, D), jnp.float32),
            pltpu.VMEM_SHARED((NPAD,), jnp.float32),
            pltpu.SemaphoreType.DMA,
            pltpu.SemaphoreType.DMA,
        ],
    )
    def agg(h_hbm, src_hbm, dst_hbm, zrows_hbm, zc_hbm, ones_hbm,
            sum_hbm, cnt_hbm,
            src_v, dst_v, rows_v, rows2_v, ones_v, cz_v, acc_s, cntacc_s,
            sem0, sem1):
        cid = lax.axis_index("c")
        sid = lax.axis_index("s")
        wid = sid * NC + cid
        pltpu.sync_copy(ones_hbm, ones_v)
        pltpu.sync_copy(zc_hbm, cz_v)
        pltpu.sync_copy(zrows_hbm, rows_v)
        for k in range(NKW):
            sl = pl.ds(sid * RPT + k * CHUNK, CHUNK)
            pltpu.sync_copy(rows_v, acc_s.at[sl])
        pltpu.sync_copy(cz_v, cntacc_s.at[pl.ds(sid * RPT, RPT)])
        plsc.subcore_barrier()

        bufs = (rows_v, rows2_v)
        sems = (sem0, sem1)

        def block(b, carry):
            pltpu.sync_copy(src_hbm.at[wid, pl.ds(b * IBLK, IBLK)], src_v)
            pltpu.sync_copy(dst_hbm.at[wid, pl.ds(b * IBLK, IBLK)], dst_v)
            pltpu.async_copy(h_hbm.at[src_v.at[0]], bufs[0], sems[0])
            for j in range(IBLK):
                p = j % 2
                if j + 1 < IBLK:
                    q = (j + 1) % 2
                    pltpu.async_copy(h_hbm.at[src_v.at[j + 1]], bufs[q],
                                     sems[q])
                pltpu.make_async_copy(h_hbm.at[src_v.at[j]], bufs[p],
                                      sems[p]).wait()
                pltpu.sync_copy(bufs[p], acc_s.at[dst_v.at[j]], add=True)
                pltpu.sync_copy(ones_v, cntacc_s.at[dst_v.at[j]], add=True)
            return carry

        lax.fori_loop(0, nch // IBLK, block, 0)
        plsc.subcore_barrier()
        for k in range(NKW):
            sl = pl.ds(sid * RPT + k * CHUNK, CHUNK)
            pltpu.sync_copy(acc_s.at[sl], rows_v)
            pltpu.sync_copy(rows_v, sum_hbm.at[cid, sl])
        slc = pl.ds(sid * RPT, RPT)
        pltpu.sync_copy(cntacc_s.at[slc], cz_v)
        pltpu.sync_copy(cz_v, cnt_hbm.at[cid, slc])

    return agg


def _dense_sage(sums, cnt, h, Wl, bl, Wr):
    BR = 2000
    grid = (N // BR,)

    def body(s0_ref, s1_ref, cnt_ref, h_ref, Wl_ref, bl_ref, Wr_ref,
             o_ref):
        denom = jnp.maximum(cnt_ref[...], 1.0)
        mean = (s0_ref[0] + s1_ref[0]) / denom
        out = (jnp.dot(mean, Wl_ref[...], preferred_element_type=jnp.float32)
               + jnp.dot(h_ref[...], Wr_ref[...],
                         preferred_element_type=jnp.float32)
               + bl_ref[...])
        o_ref[...] = jnp.maximum(out, 0.0)

    rowspec = pl.BlockSpec((BR, D), lambda i: (i, 0))
    p0spec = pl.BlockSpec((1, BR, D), lambda i: (0, i, 0))
    p1spec = pl.BlockSpec((1, BR, D), lambda i: (1, i, 0))
    cntspec = pl.BlockSpec((BR, 1), lambda i: (i, 0))
    wspec = pl.BlockSpec((D, D), lambda i: (0, 0))
    bspec = pl.BlockSpec((1, D), lambda i: (0, 0))
    return pl.pallas_call(
        body, grid=grid,
        in_specs=[p0spec, p1spec, cntspec, rowspec,
                  wspec, bspec, wspec],
        out_specs=rowspec,
        out_shape=jax.ShapeDtypeStruct((N, D), jnp.float32),
    )(sums, sums, cnt, h, Wl, bl.reshape(1, D), Wr)


def _dense_final(sums, cnt, h, Wl, bl, Wr, Wm1p, bm1p, Wm2p, bm2):
    BR = 2000
    grid = (N // BR,)

    def body(s0_ref, s1_ref, cnt_ref, h_ref, Wl_ref, bl_ref, Wr_ref,
             Wm1_ref, bm1_ref, Wm2_ref, bm2_ref, emb_ref, pred_ref):
        denom = jnp.maximum(cnt_ref[...], 1.0)
        mean = (s0_ref[0] + s1_ref[0]) / denom
        emb = (jnp.dot(mean, Wl_ref[...], preferred_element_type=jnp.float32)
               + jnp.dot(h_ref[...], Wr_ref[...],
                         preferred_element_type=jnp.float32)
               + bl_ref[...])
        emb_ref[...] = emb
        h1 = jnp.maximum(emb, 0.0)
        z = jnp.maximum(
            jnp.dot(h1, Wm1_ref[...], preferred_element_type=jnp.float32)
            + bm1_ref[...], 0.0)
        pred_ref[...] = (
            jnp.dot(z, Wm2_ref[...], preferred_element_type=jnp.float32)
            + bm2_ref[...])

    rowspec = pl.BlockSpec((BR, D), lambda i: (i, 0))
    p0spec = pl.BlockSpec((1, BR, D), lambda i: (0, i, 0))
    p1spec = pl.BlockSpec((1, BR, D), lambda i: (1, i, 0))
    cntspec = pl.BlockSpec((BR, 1), lambda i: (i, 0))
    wspec = pl.BlockSpec((D, D), lambda i: (0, 0))
    bspec = pl.BlockSpec((1, D), lambda i: (0, 0))
    sspec = pl.BlockSpec((1, 1), lambda i: (0, 0))
    return pl.pallas_call(
        body, grid=grid,
        in_specs=[p0spec, p1spec, cntspec, rowspec,
                  wspec, bspec, wspec, wspec, bspec, wspec, sspec],
        out_specs=[rowspec, rowspec],
        out_shape=[jax.ShapeDtypeStruct((N, D), jnp.float32),
                   jax.ShapeDtypeStruct((N, D), jnp.float32)],
    )(sums, sums, cnt, h, Wl, bl.reshape(1, D), Wr,
      Wm1p, bm1p.reshape(1, D), Wm2p, bm2.reshape(1, 1))


def kernel(x, edge_index, Wl0, bl0, Wr0, Wl1, bl1, Wr1, Wm1, bm1, Wm2, bm2):
    E = edge_index.shape[1]
    src = edge_index[0].astype(jnp.int32)
    dst = edge_index[1].astype(jnp.int32)
    nch = -(-(-(-E // (NW * CHUNK))) // IBLK) * IBLK
    epw = nch * CHUNK
    pad = NW * epw - E
    pad_src = (jnp.arange(pad, dtype=jnp.int32) * 61) % N
    pad_dst = N + (jnp.arange(pad, dtype=jnp.int32) % L)
    srcp = jnp.concatenate([src, pad_src]).reshape(NW, nch, CHUNK)
    dstp = jnp.concatenate([dst, pad_dst]).reshape(NW, nch, CHUNK)

    zrows = jnp.zeros((CHUNK, D), jnp.float32)
    zc = jnp.zeros((RPT,), jnp.float32)
    ones = jnp.ones((CHUNK,), jnp.float32)
    Wm1p = jnp.pad(Wm1, ((0, 0), (0, D - Wm1.shape[1])))
    bm1p = jnp.pad(bm1, (0, D - bm1.shape[0]))
    Wm2p = jnp.pad(Wm2, ((0, D - Wm2.shape[0]), (0, D - Wm2.shape[1])))

    agg = _make_agg(nch)
    sums, cnts = agg(x, srcp, dstp, zrows, zc, ones)
    cnt = (cnts[0, :N] + cnts[1, :N]).reshape(N, 1)
    h0 = _dense_sage(sums, cnt, x, Wl0, bl0, Wr0)
    sums1, _ = agg(h0, srcp, dstp, zrows, zc, ones)
    emb, pred_wide = _dense_final(sums1, cnt, h0, Wl1, bl1, Wr1,
                                  Wm1p, bm1p, Wm2p, bm2)
    return emb, pred_wide[:, :1]


# speedup vs baseline: 12.3576x; 1.0014x over previous
import functools

import jax
import jax.numpy as jnp
from jax import lax
from jax.experimental import pallas as pl
from jax.experimental.pallas import tpu as pltpu
from jax.experimental.pallas import tpu_sc as plsc

N = 10000
D = 128
NC = 2
NS = 16
L = 16
NW = NC * NS
CHUNK = 128
IBLK = 16
NPAD = 10240
RPT = NPAD // NS
NKW = RPT // CHUNK


@functools.cache
def _make_agg(nch):
    mesh = plsc.VectorSubcoreMesh(
        core_axis_name="c", subcore_axis_name="s",
        num_cores=NC, num_subcores=NS)

    @functools.partial(
        pl.kernel,
        out_type=[
            jax.ShapeDtypeStruct((NC, NPAD, D), jnp.float32),
            jax.ShapeDtypeStruct((NC, NPAD), jnp.float32),
        ],
        mesh=mesh,
        scratch_types=[
            pltpu.VMEM((IBLK, CHUNK), jnp.int32),
            pltpu.VMEM((IBLK, CHUNK), jnp.int32),
            pltpu.VMEM((CHUNK, D), jnp.float32),
            pltpu.VMEM((CHUNK, D), jnp.float32),
            pltpu.VMEM((CHUNK,), jnp.float32),
            pltpu.VMEM((RPT,), jnp.float32),
            pltpu.VMEM_SHARED((NPAD, D), jnp.float32),
            pltpu.VMEM_SHARED((NPAD,), jnp.float32),
            pltpu.SemaphoreType.DMA,
            pltpu.SemaphoreType.DMA,
            pltpu.SemaphoreType.DMA,
        ],
    )
    def agg(h_hbm, src_hbm, dst_hbm, zrows_hbm, zc_hbm, ones_hbm,
            sum_hbm, cnt_hbm,
            src_v, dst_v, rows_v, rows2_v, ones_v, cz_v, acc_s, cntacc_s,
            sem0, sem1, csem):
        cid = lax.axis_index("c")
        sid = lax.axis_index("s")
        wid = sid * NC + cid
        pltpu.sync_copy(ones_hbm, ones_v)
        pltpu.sync_copy(zc_hbm, cz_v)
        pltpu.sync_copy(zrows_hbm, rows_v)
        for k in range(NKW):
            sl = pl.ds(sid * RPT + k * CHUNK, CHUNK)
            pltpu.sync_copy(rows_v, acc_s.at[sl])
        pltpu.sync_copy(cz_v, cntacc_s.at[pl.ds(sid * RPT, RPT)])
        plsc.subcore_barrier()

        bufs = (rows_v, rows2_v)
        sems = (sem0, sem1)

        def block(b, carry):
            pltpu.sync_copy(src_hbm.at[wid, pl.ds(b * IBLK, IBLK)], src_v)
            pltpu.sync_copy(dst_hbm.at[wid, pl.ds(b * IBLK, IBLK)], dst_v)
            pltpu.async_copy(h_hbm.at[src_v.at[0]], bufs[0], sems[0])
            for j in range(IBLK):
                p = j % 2
                if j + 1 < IBLK:
                    q = (j + 1) % 2
                    pltpu.async_copy(h_hbm.at[src_v.at[j + 1]], bufs[q],
                                     sems[q])
                pltpu.make_async_copy(h_hbm.at[src_v.at[j]], bufs[p],
                                      sems[p]).wait()
                if j >= 1:
                    pltpu.make_async_copy(ones_v,
                                          cntacc_s.at[dst_v.at[j - 1]],
                                          csem).wait()
                pltpu.async_copy(ones_v, cntacc_s.at[dst_v.at[j]], csem,
                                 add=True)
                pltpu.sync_copy(bufs[p], acc_s.at[dst_v.at[j]], add=True)
            pltpu.make_async_copy(ones_v, cntacc_s.at[dst_v.at[IBLK - 1]],
                                  csem).wait()
            return carry

        lax.fori_loop(0, nch // IBLK, block, 0)
        plsc.subcore_barrier()
        for k in range(NKW):
            sl = pl.ds(sid * RPT + k * CHUNK, CHUNK)
            pltpu.sync_copy(acc_s.at[sl], rows_v)
            pltpu.sync_copy(rows_v, sum_hbm.at[cid, sl])
        slc = pl.ds(sid * RPT, RPT)
        pltpu.sync_copy(cntacc_s.at[slc], cz_v)
        pltpu.sync_copy(cz_v, cnt_hbm.at[cid, slc])

    return agg


def _dense_sage(sums, cnt, h, Wl, bl, Wr):
    BR = 2000
    grid = (N // BR,)

    def body(s0_ref, s1_ref, cnt_ref, h_ref, Wl_ref, bl_ref, Wr_ref,
             o_ref):
        denom = jnp.maximum(cnt_ref[...], 1.0)
        mean = (s0_ref[0] + s1_ref[0]) / denom
        out = (jnp.dot(mean, Wl_ref[...], preferred_element_type=jnp.float32)
               + jnp.dot(h_ref[...], Wr_ref[...],
                         preferred_element_type=jnp.float32)
               + bl_ref[...])
        o_ref[...] = jnp.maximum(out, 0.0)

    rowspec = pl.BlockSpec((BR, D), lambda i: (i, 0))
    p0spec = pl.BlockSpec((1, BR, D), lambda i: (0, i, 0))
    p1spec = pl.BlockSpec((1, BR, D), lambda i: (1, i, 0))
    cntspec = pl.BlockSpec((BR, 1), lambda i: (i, 0))
    wspec = pl.BlockSpec((D, D), lambda i: (0, 0))
    bspec = pl.BlockSpec((1, D), lambda i: (0, 0))
    return pl.pallas_call(
        body, grid=grid,
        in_specs=[p0spec, p1spec, cntspec, rowspec,
                  wspec, bspec, wspec],
        out_specs=rowspec,
        out_shape=jax.ShapeDtypeStruct((N, D), jnp.float32),
    )(sums, sums, cnt, h, Wl, bl.reshape(1, D), Wr)


def _dense_final(sums, cnt, h, Wl, bl, Wr, Wm1p, bm1p, Wm2p, bm2):
    BR = 2000
    grid = (N // BR,)

    def body(s0_ref, s1_ref, cnt_ref, h_ref, Wl_ref, bl_ref, Wr_ref,
             Wm1_ref, bm1_ref, Wm2_ref, bm2_ref, emb_ref, pred_ref):
        denom = jnp.maximum(cnt_ref[...], 1.0)
        mean = (s0_ref[0] + s1_ref[0]) / denom
        emb = (jnp.dot(mean, Wl_ref[...], preferred_element_type=jnp.float32)
               + jnp.dot(h_ref[...], Wr_ref[...],
                         preferred_element_type=jnp.float32)
               + bl_ref[...])
        emb_ref[...] = emb
        h1 = jnp.maximum(emb, 0.0)
        z = jnp.maximum(
            jnp.dot(h1, Wm1_ref[...], preferred_element_type=jnp.float32)
            + bm1_ref[...], 0.0)
        pred_ref[...] = (
            jnp.dot(z, Wm2_ref[...], preferred_element_type=jnp.float32)
            + bm2_ref[...])

    rowspec = pl.BlockSpec((BR, D), lambda i: (i, 0))
    p0spec = pl.BlockSpec((1, BR, D), lambda i: (0, i, 0))
    p1spec = pl.BlockSpec((1, BR, D), lambda i: (1, i, 0))
    cntspec = pl.BlockSpec((BR, 1), lambda i: (i, 0))
    wspec = pl.BlockSpec((D, D), lambda i: (0, 0))
    bspec = pl.BlockSpec((1, D), lambda i: (0, 0))
    sspec = pl.BlockSpec((1, 1), lambda i: (0, 0))
    return pl.pallas_call(
        body, grid=grid,
        in_specs=[p0spec, p1spec, cntspec, rowspec,
                  wspec, bspec, wspec, wspec, bspec, wspec, sspec],
        out_specs=[rowspec, rowspec],
        out_shape=[jax.ShapeDtypeStruct((N, D), jnp.float32),
                   jax.ShapeDtypeStruct((N, D), jnp.float32)],
    )(sums, sums, cnt, h, Wl, bl.reshape(1, D), Wr,
      Wm1p, bm1p.reshape(1, D), Wm2p, bm2.reshape(1, 1))


def kernel(x, edge_index, Wl0, bl0, Wr0, Wl1, bl1, Wr1, Wm1, bm1, Wm2, bm2):
    E = edge_index.shape[1]
    src = edge_index[0].astype(jnp.int32)
    dst = edge_index[1].astype(jnp.int32)
    nch = -(-(-(-E // (NW * CHUNK))) // IBLK) * IBLK
    epw = nch * CHUNK
    pad = NW * epw - E
    pad_src = (jnp.arange(pad, dtype=jnp.int32) * 61) % N
    pad_dst = N + (jnp.arange(pad, dtype=jnp.int32) % L)
    srcp = jnp.concatenate([src, pad_src]).reshape(NW, nch, CHUNK)
    dstp = jnp.concatenate([dst, pad_dst]).reshape(NW, nch, CHUNK)

    zrows = jnp.zeros((CHUNK, D), jnp.float32)
    zc = jnp.zeros((RPT,), jnp.float32)
    ones = jnp.ones((CHUNK,), jnp.float32)
    Wm1p = jnp.pad(Wm1, ((0, 0), (0, D - Wm1.shape[1])))
    bm1p = jnp.pad(bm1, (0, D - bm1.shape[0]))
    Wm2p = jnp.pad(Wm2, ((0, D - Wm2.shape[0]), (0, D - Wm2.shape[1])))

    agg = _make_agg(nch)
    sums, cnts = agg(x, srcp, dstp, zrows, zc, ones)
    cnt = (cnts[0, :N] + cnts[1, :N]).reshape(N, 1)
    h0 = _dense_sage(sums, cnt, x, Wl0, bl0, Wr0)
    sums1, _ = agg(h0, srcp, dstp, zrows, zc, ones)
    emb, pred_wide = _dense_final(sums1, cnt, h0, Wl1, bl1, Wr1,
                                  Wm1p, bm1p, Wm2p, bm2)
    return emb, pred_wide[:, :1]
